# baseline (speedup 1.0000x reference)
import functools

import numpy as np
import jax
import jax.numpy as jnp
from jax import lax
from jax.experimental import pallas as pl
from jax.experimental.pallas import tpu as pltpu
from jax.experimental.pallas import tpu_sc as plsc

N = 10000
E = 160000
DE = 16
H = 32
HH = H * H

NC = 2
NS = 16
NW = NC * NS
CH = 128
NCHUNK = E // CH
CHUNK_PER_W = -(-NCHUNK // NW)
RS_MAIN = 640
RS_LAST = N - RS_MAIN * (NS - 1)

BB = 4000
NBLK = E // BB

_ar = np.arange(HH)
_R = np.zeros((128, HH), np.float32)
_R[:H] = (np.arange(H)[:, None] == (_ar[None, :] // H)).astype(np.float32)
_S128 = np.zeros((HH, 128), np.float32)
_S128[:, :H] = ((_ar[:, None] % H) == np.arange(H)[None, :]).astype(np.float32)
_ONESROW = np.zeros((1, 128), np.float32)
_ONESROW[0, H] = 1.0


def _gather_xs(x128, src1d, ne):
    mesh = plsc.VectorSubcoreMesh(core_axis_name="core", subcore_axis_name="subcore")

    @functools.partial(
        pl.kernel,
        out_type=jax.ShapeDtypeStruct((ne, 128), jnp.float32),
        mesh=mesh,
        scratch_types=[
            pltpu.VMEM((CH,), jnp.int32),
            pltpu.VMEM((CH, 128), jnp.float32),
            pltpu.VMEM_SHARED((N, 128), jnp.float32),
        ],
    )
    def gather_kernel(x_hbm, i_hbm, o_hbm, idx_v, row_v, x_s):
        cid = lax.axis_index("core")
        sid = lax.axis_index("subcore")
        wid = cid * NS + sid
        r0 = pl.multiple_of(sid * RS_MAIN, 8)

        @pl.when(sid < NS - 1)
        def _():
            pltpu.sync_copy(x_hbm.at[pl.ds(r0, RS_MAIN)],
                            x_s.at[pl.ds(r0, RS_MAIN)])

        @pl.when(sid == NS - 1)
        def _():
            pltpu.sync_copy(x_hbm.at[pl.ds(r0, RS_LAST)],
                            x_s.at[pl.ds(r0, RS_LAST)])

        plsc.subcore_barrier()

        nchunk = ne // CH

        @pl.loop(0, -(-nchunk // NW))
        def _(t):
            c = wid + t * NW

            @pl.when(c < nchunk)
            def _():
                e0 = pl.multiple_of(c * CH, CH)
                pltpu.sync_copy(i_hbm.at[pl.ds(e0, CH)], idx_v)
                pltpu.sync_copy(x_s.at[idx_v], row_v)
                pltpu.sync_copy(row_v, o_hbm.at[pl.ds(e0, CH)])

    return gather_kernel(x128, src1d)


def _scatter_mean_parts(msg, dst1d, zeros_a, ne):
    mesh = plsc.VectorSubcoreMesh(core_axis_name="core", subcore_axis_name="subcore")

    @functools.partial(
        pl.kernel,
        out_type=jax.ShapeDtypeStruct((NC, N, 128), jnp.float32),
        mesh=mesh,
        scratch_types=[
            pltpu.VMEM((CH,), jnp.int32),
            pltpu.VMEM((CH, 128), jnp.float32),
            pltpu.VMEM_SHARED((N, 128), jnp.float32),
        ],
    )
    def scatter_kernel(msg_hbm, dst_hbm, za_hbm, agg_hbm, idx_v, row_v, agg_s):
        cid = lax.axis_index("core")
        sid = lax.axis_index("subcore")
        wid = cid * NS + sid

        r0 = pl.multiple_of(sid * RS_MAIN, 8)

        @pl.when(sid < NS - 1)
        def _():
            pltpu.sync_copy(za_hbm.at[pl.ds(r0, RS_MAIN)],
                            agg_s.at[pl.ds(r0, RS_MAIN)])

        @pl.when(sid == NS - 1)
        def _():
            pltpu.sync_copy(za_hbm.at[pl.ds(r0, RS_LAST)],
                            agg_s.at[pl.ds(r0, RS_LAST)])

        plsc.subcore_barrier()

        nchunk = ne // CH

        @pl.loop(0, -(-nchunk // NW))
        def _(t):
            c = wid + t * NW

            @pl.when(c < nchunk)
            def _():
                e0 = pl.multiple_of(c * CH, CH)
                pltpu.sync_copy(dst_hbm.at[pl.ds(e0, CH)], idx_v)
                pltpu.sync_copy(msg_hbm.at[pl.ds(e0, CH)], row_v)
                pltpu.sync_copy(row_v, agg_s.at[idx_v], add=True)

        plsc.subcore_barrier()

        @pl.when(sid < NS - 1)
        def _():
            pltpu.sync_copy(agg_s.at[pl.ds(r0, RS_MAIN)],
                            agg_hbm.at[cid, pl.ds(r0, RS_MAIN)])

        @pl.when(sid == NS - 1)
        def _():
            pltpu.sync_copy(agg_s.at[pl.ds(r0, RS_LAST)],
                            agg_hbm.at[cid, pl.ds(r0, RS_LAST)])

    return scatter_kernel(msg, dst1d, zeros_a)


def _mlp_body(ea_ref, xs_ref, w1_ref, b1_ref, w2_ref, b2_ref, w3_ref, b3_ref,
              rg_ref, s_ref, p_ref, q_ref, ones_ref, out_ref):
    bf = jnp.bfloat16
    h = jnp.maximum(jnp.dot(ea_ref[...].astype(bf), w1_ref[...],
                            preferred_element_type=jnp.float32) + b1_ref[...], 0.0)
    h = jnp.maximum(jnp.dot(h.astype(bf), w2_ref[...],
                            preferred_element_type=jnp.float32) + b2_ref[...], 0.0)
    h3 = jnp.dot(h.astype(bf), w3_ref[...],
                 preferred_element_type=jnp.float32) + b3_ref[...]
    mu = jnp.mean(h3, axis=1, keepdims=True)
    ms = jnp.mean(h3 * h3, axis=1, keepdims=True)
    rstd = lax.rsqrt(jnp.maximum(ms - mu * mu, 0.0) + 1e-5)
    xsb = xs_ref[...].astype(bf)
    xsg = jnp.dot(xsb, rg_ref[...], preferred_element_type=jnp.float32)
    u = (h3 * xsg).astype(bf)
    m1 = jnp.dot(u, s_ref[...], preferred_element_type=jnp.float32)
    t1 = jnp.dot(xsb, p_ref[...], preferred_element_type=jnp.float32)
    t2 = jnp.dot(xsb, q_ref[...], preferred_element_type=jnp.float32)
    out_ref[...] = m1 * rstd - (mu * rstd) * t1 + t2 + ones_ref[...]


def _edge_messages(edge_attr, xs, W1, b1, W2, b2, W3, b3, ln_g, ln_b, ne):
    S128 = jnp.asarray(_S128)
    Rg = jnp.asarray(_R) * ln_g.reshape(1, -1)
    P = Rg @ S128
    Q = (jnp.asarray(_R) * ln_b.reshape(1, -1)) @ S128
    full = lambda shape: pl.BlockSpec(shape, lambda i: tuple(0 for _ in shape))
    return pl.pallas_call(
        _mlp_body,
        grid=(ne // BB,),
        in_specs=[
            pl.BlockSpec((BB, DE), lambda i: (i, 0)),
            pl.BlockSpec((BB, 128), lambda i: (i, 0)),
            full((DE, 128)), full((1, 128)),
            full((128, 128)), full((1, 128)),
            full((128, HH)), full((1, HH)),
            full((128, HH)), full((HH, 128)),
            full((128, 128)), full((128, 128)), full((1, 128)),
        ],
        out_specs=pl.BlockSpec((BB, 128), lambda i: (i, 0)),
        out_shape=jax.ShapeDtypeStruct((ne, 128), jnp.float32),
    )(edge_attr, xs, W1.astype(jnp.bfloat16), b1.reshape(1, -1),
      W2.astype(jnp.bfloat16), b2.reshape(1, -1),
      W3.astype(jnp.bfloat16), b3.reshape(1, -1),
      Rg.astype(jnp.bfloat16), jnp.asarray(_S128).astype(jnp.bfloat16),
      P.astype(jnp.bfloat16), Q.astype(jnp.bfloat16), jnp.asarray(_ONESROW))


def _finish_body(x_ref, agg_ref, root_ref, cb_ref, cng_ref,
                 cnb_ref, out_ref):
    a = agg_ref[0] + agg_ref[1]
    agg = a[:, :H]
    cnt = a[:, H:H + 1]
    agg = agg / jnp.maximum(cnt, 1.0)
    pre = agg + jnp.dot(x_ref[...], root_ref[...],
                        preferred_element_type=jnp.float32) + cb_ref[...]
    out = jnp.maximum(pre, 0.0)
    gm = jnp.mean(out)
    d = out - gm
    gv = jnp.mean(d * d)
    out_ref[...] = x_ref[...] + d * lax.rsqrt(gv + 1e-5) * cng_ref[...] + cnb_ref[...]


def _finish(x, agg_parts, root, conv_bias, cn_g, cn_b):
    full = lambda shape: pl.BlockSpec(shape, lambda *_: tuple(0 for _ in shape))
    return pl.pallas_call(
        _finish_body,
        grid=(1,),
        in_specs=[
            full((N, H)), full((NC, N, 128)),
            full((H, H)), full((1, H)), full((1, H)), full((1, H)),
        ],
        out_specs=full((N, H)),
        out_shape=jax.ShapeDtypeStruct((N, H), jnp.float32),
    )(x, agg_parts, root, conv_bias.reshape(1, -1),
      cn_g.reshape(1, -1), cn_b.reshape(1, -1))


def kernel(x, edge_index, edge_attr, W1, b1, W2, b2, W3, b3, ln_g, ln_b,
           root, conv_bias, cn_g, cn_b):
    src1d = edge_index[0].reshape(E)
    dst1d = edge_index[1].reshape(E)
    x128 = jnp.pad(x, ((0, 0), (0, 128 - H)))
    zeros_a = jnp.zeros((N, 128), jnp.float32)
    K = 5
    EH = E // K
    xs = [_gather_xs(x128, src1d[k * EH:(k + 1) * EH], EH) for k in range(K)]
    msgs = [_edge_messages(edge_attr[k * EH:(k + 1) * EH], xs[k],
                           W1, b1, W2, b2, W3, b3, ln_g, ln_b, EH)
            for k in range(K)]
    aggs = [_scatter_mean_parts(msgs[k], dst1d[k * EH:(k + 1) * EH],
                                zeros_a, EH) for k in range(K)]
    total = aggs[0]
    for a in aggs[1:]:
        total = total + a
    return _finish(x, total, root, conv_bias, cn_g, cn_b)


# speedup vs baseline: 1.0681x; 1.0681x over previous
import functools

import numpy as np
import jax
import jax.numpy as jnp
from jax import lax
from jax.experimental import pallas as pl
from jax.experimental.pallas import tpu as pltpu
from jax.experimental.pallas import tpu_sc as plsc

N = 10000
E = 160000
DE = 16
H = 32
HH = H * H

NC = 2
NS = 16
NW = NC * NS
CH = 128
NCHUNK = E // CH
CHUNK_PER_W = -(-NCHUNK // NW)
RS_MAIN = 640
RS_LAST = N - RS_MAIN * (NS - 1)

BB = 4000
NBLK = E // BB

_ar = np.arange(HH)
_R = np.zeros((128, HH), np.float32)
_R[:H] = (np.arange(H)[:, None] == (_ar[None, :] // H)).astype(np.float32)
_S128 = np.zeros((HH, 128), np.float32)
_S128[:, :H] = ((_ar[:, None] % H) == np.arange(H)[None, :]).astype(np.float32)
_ONESROW = np.zeros((1, 128), np.float32)
_ONESROW[0, H] = 1.0


def _gather_xs(x128, src1d, ne):
    mesh = plsc.VectorSubcoreMesh(core_axis_name="core", subcore_axis_name="subcore")

    @functools.partial(
        pl.kernel,
        out_type=jax.ShapeDtypeStruct((ne, 128), jnp.float32),
        mesh=mesh,
        scratch_types=[
            pltpu.VMEM((CH,), jnp.int32),
            pltpu.VMEM((CH, 128), jnp.float32),
            pltpu.VMEM_SHARED((N, 128), jnp.float32),
        ],
    )
    def gather_kernel(x_hbm, i_hbm, o_hbm, idx_v, row_v, x_s):
        cid = lax.axis_index("core")
        sid = lax.axis_index("subcore")
        wid = cid * NS + sid
        r0 = pl.multiple_of(sid * RS_MAIN, 8)

        @pl.when(sid < NS - 1)
        def _():
            pltpu.sync_copy(x_hbm.at[pl.ds(r0, RS_MAIN)],
                            x_s.at[pl.ds(r0, RS_MAIN)])

        @pl.when(sid == NS - 1)
        def _():
            pltpu.sync_copy(x_hbm.at[pl.ds(r0, RS_LAST)],
                            x_s.at[pl.ds(r0, RS_LAST)])

        plsc.subcore_barrier()

        nchunk = ne // CH

        @pl.loop(0, -(-nchunk // NW))
        def _(t):
            c = wid + t * NW

            @pl.when(c < nchunk)
            def _():
                e0 = pl.multiple_of(c * CH, CH)
                pltpu.sync_copy(i_hbm.at[pl.ds(e0, CH)], idx_v)
                pltpu.sync_copy(x_s.at[idx_v], row_v)
                pltpu.sync_copy(row_v, o_hbm.at[pl.ds(e0, CH)])

    return gather_kernel(x128, src1d)


def _scatter_mean_parts(msg, dst1d, zeros_a, ne):
    mesh = plsc.VectorSubcoreMesh(core_axis_name="core", subcore_axis_name="subcore")

    @functools.partial(
        pl.kernel,
        out_type=jax.ShapeDtypeStruct((NC, N, 128), jnp.float32),
        mesh=mesh,
        scratch_types=[
            pltpu.VMEM((CH,), jnp.int32),
            pltpu.VMEM((CH, 128), jnp.float32),
            pltpu.VMEM_SHARED((N, 128), jnp.float32),
        ],
    )
    def scatter_kernel(msg_hbm, dst_hbm, za_hbm, agg_hbm, idx_v, row_v, agg_s):
        cid = lax.axis_index("core")
        sid = lax.axis_index("subcore")
        wid = cid * NS + sid

        r0 = pl.multiple_of(sid * RS_MAIN, 8)

        @pl.when(sid < NS - 1)
        def _():
            pltpu.sync_copy(za_hbm.at[cid, pl.ds(r0, RS_MAIN)],
                            agg_s.at[pl.ds(r0, RS_MAIN)])

        @pl.when(sid == NS - 1)
        def _():
            pltpu.sync_copy(za_hbm.at[cid, pl.ds(r0, RS_LAST)],
                            agg_s.at[pl.ds(r0, RS_LAST)])

        plsc.subcore_barrier()

        nchunk = ne // CH

        @pl.loop(0, -(-nchunk // NW))
        def _(t):
            c = wid + t * NW

            @pl.when(c < nchunk)
            def _():
                e0 = pl.multiple_of(c * CH, CH)
                pltpu.sync_copy(dst_hbm.at[pl.ds(e0, CH)], idx_v)
                pltpu.sync_copy(msg_hbm.at[pl.ds(e0, CH)], row_v)
                pltpu.sync_copy(row_v, agg_s.at[idx_v], add=True)

        plsc.subcore_barrier()

        @pl.when(sid < NS - 1)
        def _():
            pltpu.sync_copy(agg_s.at[pl.ds(r0, RS_MAIN)],
                            agg_hbm.at[cid, pl.ds(r0, RS_MAIN)])

        @pl.when(sid == NS - 1)
        def _():
            pltpu.sync_copy(agg_s.at[pl.ds(r0, RS_LAST)],
                            agg_hbm.at[cid, pl.ds(r0, RS_LAST)])

    return scatter_kernel(msg, dst1d, zeros_a)


def _mlp_body(ea_ref, xs_ref, w1_ref, b1_ref, w2_ref, b2_ref, w3_ref, b3_ref,
              rg_ref, s_ref, p_ref, q_ref, ones_ref, out_ref):
    bf = jnp.bfloat16
    h = jnp.maximum(jnp.dot(ea_ref[...].astype(bf), w1_ref[...],
                            preferred_element_type=jnp.float32) + b1_ref[...], 0.0)
    h = jnp.maximum(jnp.dot(h.astype(bf), w2_ref[...],
                            preferred_element_type=jnp.float32) + b2_ref[...], 0.0)
    h3 = jnp.dot(h.astype(bf), w3_ref[...],
                 preferred_element_type=jnp.float32) + b3_ref[...]
    mu = jnp.mean(h3, axis=1, keepdims=True)
    ms = jnp.mean(h3 * h3, axis=1, keepdims=True)
    rstd = lax.rsqrt(jnp.maximum(ms - mu * mu, 0.0) + 1e-5)
    xsb = xs_ref[...].astype(bf)
    xsg = jnp.dot(xsb, rg_ref[...], preferred_element_type=jnp.float32)
    u = (h3 * xsg).astype(bf)
    m1 = jnp.dot(u, s_ref[...], preferred_element_type=jnp.float32)
    t1 = jnp.dot(xsb, p_ref[...], preferred_element_type=jnp.float32)
    t2 = jnp.dot(xsb, q_ref[...], preferred_element_type=jnp.float32)
    out_ref[...] = m1 * rstd - (mu * rstd) * t1 + t2 + ones_ref[...]


def _edge_messages(edge_attr, xs, W1, b1, W2, b2, W3, b3, ln_g, ln_b, ne, blk0):
    S128 = jnp.asarray(_S128)
    Rg = jnp.asarray(_R) * ln_g.reshape(1, -1)
    P = Rg @ S128
    Q = (jnp.asarray(_R) * ln_b.reshape(1, -1)) @ S128
    full = lambda shape: pl.BlockSpec(shape, lambda i: tuple(0 for _ in shape))
    return pl.pallas_call(
        _mlp_body,
        grid=(ne // BB,),
        in_specs=[
            pl.BlockSpec((BB, DE), lambda i: (i + blk0, 0)),
            pl.BlockSpec((BB, 128), lambda i: (i, 0)),
            full((DE, 128)), full((1, 128)),
            full((128, 128)), full((1, 128)),
            full((128, HH)), full((1, HH)),
            full((128, HH)), full((HH, 128)),
            full((128, 128)), full((128, 128)), full((1, 128)),
        ],
        out_specs=pl.BlockSpec((BB, 128), lambda i: (i, 0)),
        out_shape=jax.ShapeDtypeStruct((ne, 128), jnp.float32),
    )(edge_attr, xs, W1.astype(jnp.bfloat16), b1.reshape(1, -1),
      W2.astype(jnp.bfloat16), b2.reshape(1, -1),
      W3.astype(jnp.bfloat16), b3.reshape(1, -1),
      Rg.astype(jnp.bfloat16), jnp.asarray(_S128).astype(jnp.bfloat16),
      P.astype(jnp.bfloat16), Q.astype(jnp.bfloat16), jnp.asarray(_ONESROW))


def _finish_body(x_ref, agg_ref, root_ref, cb_ref, cng_ref,
                 cnb_ref, out_ref):
    a = agg_ref[0] + agg_ref[1]
    agg = a[:, :H]
    cnt = a[:, H:H + 1]
    agg = agg / jnp.maximum(cnt, 1.0)
    pre = agg + jnp.dot(x_ref[...], root_ref[...],
                        preferred_element_type=jnp.float32) + cb_ref[...]
    out = jnp.maximum(pre, 0.0)
    gm = jnp.mean(out)
    d = out - gm
    gv = jnp.mean(d * d)
    out_ref[...] = x_ref[...] + d * lax.rsqrt(gv + 1e-5) * cng_ref[...] + cnb_ref[...]


def _finish(x, agg_parts, root, conv_bias, cn_g, cn_b):
    full = lambda shape: pl.BlockSpec(shape, lambda *_: tuple(0 for _ in shape))
    return pl.pallas_call(
        _finish_body,
        grid=(1,),
        in_specs=[
            full((N, H)), full((NC, N, 128)),
            full((H, H)), full((1, H)), full((1, H)), full((1, H)),
        ],
        out_specs=full((N, H)),
        out_shape=jax.ShapeDtypeStruct((N, H), jnp.float32),
    )(x, agg_parts, root, conv_bias.reshape(1, -1),
      cn_g.reshape(1, -1), cn_b.reshape(1, -1))


def kernel(x, edge_index, edge_attr, W1, b1, W2, b2, W3, b3, ln_g, ln_b,
           root, conv_bias, cn_g, cn_b):
    src1d = edge_index[0].reshape(E)
    dst1d = edge_index[1].reshape(E)
    x128 = jnp.pad(x, ((0, 0), (0, 128 - H)))
    K = 5
    EH = E // K
    xs = [_gather_xs(x128, src1d[k * EH:(k + 1) * EH], EH) for k in range(K)]
    msgs = [_edge_messages(edge_attr, xs[k],
                           W1, b1, W2, b2, W3, b3, ln_g, ln_b, EH, k * (EH // BB))
            for k in range(K)]
    agg = jnp.zeros((NC, N, 128), jnp.float32)
    for k in range(K):
        agg = _scatter_mean_parts(msgs[k], dst1d[k * EH:(k + 1) * EH], agg, EH)
    return _finish(x, agg, root, conv_bias, cn_g, cn_b)
